# baseline (speedup 1.0000x reference)
import jax
import jax.numpy as jnp
from jax import lax
from jax.experimental import pallas as pl
from jax.experimental.pallas import tpu as pltpu
from jax.experimental.pallas import tpu_sc as plsc

_N = 16384
_W = 128
_ROWS = _N // _W
_NC = 1
_NS = plsc.get_sparse_core_info().num_subcores
_RPS = _ROWS // (_NC * _NS)


def _gather_body(tab_hbm, in_hbm, out_hbm, idx_v, val_v, sem):
    wid = lax.axis_index("s") * _NC + lax.axis_index("c")
    base = wid * _RPS
    pltpu.sync_copy(in_hbm.at[pl.ds(base, _RPS)], idx_v)
    copies = [
        pltpu.async_copy(tab_hbm.at[idx_v.at[j]], val_v.at[j], sem)
        for j in range(_RPS)
    ]
    for c in copies:
        c.wait()
    pltpu.sync_copy(val_v, out_hbm.at[pl.ds(base, _RPS)])


def kernel(input):
    x = input.astype(jnp.int32).reshape(_ROWS, _W)
    table = jnp.array([-1, 4, 3, 2, 1, -1, -1, -1], dtype=jnp.int32)
    sc_call = pl.kernel(
        _gather_body,
        out_type=jax.ShapeDtypeStruct((_ROWS, _W), jnp.int32),
        mesh=plsc.VectorSubcoreMesh(
            core_axis_name="c", subcore_axis_name="s", num_cores=_NC
        ),
        scratch_types=[
            pltpu.VMEM((_RPS, _W), jnp.int32),
            pltpu.VMEM((_RPS, _W), jnp.int32),
            pltpu.SemaphoreType.DMA,
        ],
    )
    return sc_call(table, x).reshape(_N).astype(input.dtype)


# speedup vs baseline: 5.2451x; 5.2451x over previous
import jax
import jax.numpy as jnp
from jax import lax
from jax.experimental import pallas as pl
from jax.experimental.pallas import tpu as pltpu
from jax.experimental.pallas import tpu_sc as plsc

_N = 16384
_LANES = 16
_NC = 1
_NS = plsc.get_sparse_core_info().num_subcores
_CHUNK = _N // (_NC * _NS)


def _lookup_body(in_hbm, out_hbm, buf):
    wid = lax.axis_index("s") * _NC + lax.axis_index("c")
    base = wid * _CHUNK
    pltpu.sync_copy(in_hbm.at[pl.ds(base, _CHUNK)], buf)

    def step(i, carry):
        x = buf[pl.ds(i * _LANES, _LANES)]
        hit = (x >= 1) & (x <= 4)
        buf[pl.ds(i * _LANES, _LANES)] = jnp.where(hit, 5 - x, -1)
        return carry

    lax.fori_loop(0, _CHUNK // _LANES, step, 0, unroll=4)
    pltpu.sync_copy(buf, out_hbm.at[pl.ds(base, _CHUNK)])


def kernel(input):
    x = input.astype(jnp.int32)
    sc_call = pl.kernel(
        _lookup_body,
        out_type=jax.ShapeDtypeStruct((_N,), jnp.int32),
        mesh=plsc.VectorSubcoreMesh(
            core_axis_name="c", subcore_axis_name="s", num_cores=_NC
        ),
        scratch_types=[pltpu.VMEM((_CHUNK,), jnp.int32)],
    )
    return sc_call(x).astype(input.dtype)
